# baseline (speedup 1.0000x reference)
import functools

import jax
import jax.numpy as jnp
from jax import lax
from jax.experimental import pallas as pl
from jax.experimental.pallas import tpu as pltpu
from jax.experimental.pallas import tpu_sc as plsc

B = 16384
D = 64
NROWS = 1000000
L = 16
NC = 2
NS = 16
NW = NC * NS
BPW = B // NW
CHUNK = 256

QROWS = 262144
QMASK = QROWS - 1
QBLK = 4096
TGRID = QROWS // QBLK


def _transpose_pack_body(x0_ref, x1_ref, x2_ref, x3_ref, y_ref):
    eye = jnp.eye(D, dtype=jnp.float32)

    for k, x_ref in enumerate((x0_ref, x1_ref, x2_ref, x3_ref)):
        y = lax.dot_general(
            x_ref[...], eye, (((0,), (0,)), ((), ())),
            precision=lax.Precision.DEFAULT)
        lo = lax.convert_element_type(
            lax.bitcast_convert_type(
                y[:, :32].astype(jnp.bfloat16), jnp.uint16), jnp.uint32)
        hi = lax.convert_element_type(
            lax.bitcast_convert_type(
                y[:, 32:].astype(jnp.bfloat16), jnp.uint16), jnp.uint32)
        packed = lax.bitcast_convert_type(
            lo | (hi << jnp.uint32(16)), jnp.int32)
        y_ref[:, 32 * k:32 * (k + 1)] = packed


def _make_tc_transpose():
    last_blk = NROWS // QBLK
    specs = [
        pl.BlockSpec(
            (D, QBLK),
            functools.partial(
                lambda k, i: (0, jnp.minimum(TGRID * k + i, last_blk)), k))
        for k in range(4)
    ]
    return pl.pallas_call(
        _transpose_pack_body,
        grid=(TGRID,),
        in_specs=specs,
        out_specs=pl.BlockSpec((QBLK, 128), lambda i: (i, 0)),
        out_shape=jax.ShapeDtypeStruct((QROWS, 128), jnp.int32),
    )


def _cosine_body(uidx_hbm, iidx_hbm, utab_hbm, itab_hbm, out_hbm,
                 uidx_v, iidx_v, urows_v, irows_v, out_v, usem, isem):
    wid = lax.axis_index("s") * NC + lax.axis_index("c")
    base = wid * BPW

    pltpu.sync_copy(uidx_hbm.at[pl.ds(base, BPW)], uidx_v)
    pltpu.sync_copy(iidx_hbm.at[pl.ds(base, BPW)], iidx_v)

    def chunk_body(ck, _):
        def fire(blk, _):
            uvec = uidx_v[pl.ds(ck * CHUNK + blk * L, L)] & QMASK
            ivec = iidx_v[pl.ds(ck * CHUNK + blk * L, L)] & QMASK
            for j in range(L):
                pltpu.async_copy(
                    utab_hbm.at[pl.ds(uvec[j], 1)],
                    urows_v.at[pl.ds(blk * L + j, 1)], usem)
                pltpu.async_copy(
                    itab_hbm.at[pl.ds(ivec[j], 1)],
                    irows_v.at[pl.ds(blk * L + j, 1)], isem)
            return 0

        lax.fori_loop(0, CHUNK // L, fire, 0)
        pltpu.make_async_copy(
            utab_hbm.at[pl.ds(0, CHUNK)], urows_v, usem).wait()
        pltpu.make_async_copy(
            itab_hbm.at[pl.ds(0, CHUNK)], irows_v, isem).wait()

        def block_body(blk, _):
            row_ids = blk * L + lax.iota(jnp.int32, L)
            sl = pl.ds(ck * CHUNK + blk * L, L)
            ucol0 = (uidx_v[sl] >> 18) * 32
            icol0 = (iidx_v[sl] >> 18) * 32
            himask = jnp.full((L,), 0xFFFF0000, jnp.uint32)
            sh16 = jnp.full((L,), 16, jnp.int32)

            def unpack(w):
                wl = plsc.bitcast(w, jnp.int32)
                a = plsc.bitcast(lax.shift_left(wl, sh16), jnp.float32)
                bbits = plsc.bitcast(wl, jnp.uint32) & himask
                b = plsc.bitcast(plsc.bitcast(bbits, jnp.int32), jnp.float32)
                return a, b

            def d_body(jj, carry):
                dot, uu, ii = carry
                uw = plsc.load_gather(urows_v, [row_ids, ucol0 + jj])
                iw = plsc.load_gather(irows_v, [row_ids, icol0 + jj])
                u0, u1 = unpack(uw)
                i0, i1 = unpack(iw)
                dot = dot + u0 * i0 + u1 * i1
                uu = uu + u0 * u0 + u1 * u1
                ii = ii + i0 * i0 + i1 * i1
                return (dot, uu, ii)

            z = jnp.zeros((L,), jnp.float32)
            dot, uu, ii = lax.fori_loop(0, 32, d_body, (z, z, z))

            p = jnp.maximum(uu * ii, 1e-30)
            bits = plsc.bitcast(p, jnp.int32)
            seed = jnp.full((L,), 0x5F3759DF, jnp.int32) - lax.shift_right_logical(
                bits, jnp.full((L,), 1, jnp.int32))
            y = plsc.bitcast(seed, jnp.float32)
            for _ in range(3):
                y = y * (1.5 - 0.5 * p * y * y)
            s = p * y
            denom = jnp.maximum(s, 1e-8)
            out_v[pl.ds(ck * CHUNK + blk * L, L)] = dot / denom
            return 0

        lax.fori_loop(0, CHUNK // L, block_body, 0)
        return 0

    lax.fori_loop(0, BPW // CHUNK, chunk_body, 0)
    pltpu.sync_copy(out_v, out_hbm.at[pl.ds(base, BPW)])


@jax.jit
def _run(query_users, query_items, user_table, item_table):
    tc_transpose = _make_tc_transpose()
    u_rows_tab = tc_transpose(*([user_table.T] * 4))
    i_rows_tab = tc_transpose(*([item_table.T] * 4))

    mesh = plsc.VectorSubcoreMesh(core_axis_name="c", subcore_axis_name="s")
    k = functools.partial(
        pl.kernel,
        mesh=mesh,
        compiler_params=pltpu.CompilerParams(needs_layout_passes=False),
        out_type=jax.ShapeDtypeStruct((B,), jnp.float32),
        scratch_types=[
            pltpu.VMEM((BPW,), jnp.int32),
            pltpu.VMEM((BPW,), jnp.int32),
            pltpu.VMEM((CHUNK, 128), jnp.int32),
            pltpu.VMEM((CHUNK, 128), jnp.int32),
            pltpu.VMEM((BPW,), jnp.float32),
            pltpu.SemaphoreType.DMA,
            pltpu.SemaphoreType.DMA,
        ],
    )(_cosine_body)
    return k(query_users, query_items, u_rows_tab, i_rows_tab)


def kernel(query_users, query_items, user_table, item_table):
    qu = query_users.astype(jnp.int32)
    qi = query_items.astype(jnp.int32)
    return _run(qu, qi, user_table, item_table)


# speedup vs baseline: 1.2715x; 1.2715x over previous
import functools

import jax
import jax.numpy as jnp
from jax import lax
from jax.experimental import pallas as pl
from jax.experimental.pallas import tpu as pltpu
from jax.experimental.pallas import tpu_sc as plsc

B = 16384
D = 64
NROWS = 1000000
L = 16
NC = 2
NS = 16
NW = NC * NS
BPW = B // NW
CHUNK = 256

QROWS = 524288
QMASK = QROWS - 1
QBLK = 4096
TGRID = QROWS // QBLK


def _transpose_body(x0_ref, x1_ref, y_ref):
    eye = jnp.eye(D, dtype=jnp.float32)
    for k, x_ref in enumerate((x0_ref, x1_ref)):
        y = lax.dot_general(
            x_ref[...], eye, (((0,), (0,)), ((), ())),
            precision=lax.Precision.DEFAULT)
        y_ref[:, D * k:D * (k + 1)] = y.astype(jnp.bfloat16)


def _make_tc_transpose():
    last_blk = NROWS // QBLK
    specs = [
        pl.BlockSpec(
            (D, QBLK),
            functools.partial(
                lambda k, i: (0, jnp.minimum(TGRID * k + i, last_blk)), k))
        for k in range(2)
    ]
    return pl.pallas_call(
        _transpose_body,
        grid=(TGRID,),
        in_specs=specs,
        out_specs=pl.BlockSpec((QBLK, 2 * D), lambda i: (i, 0)),
        out_shape=jax.ShapeDtypeStruct((QROWS, 2 * D), jnp.bfloat16),
    )


def _cosine_body(uidx_hbm, iidx_hbm, utab_hbm, itab_hbm, out_hbm,
                 uidx_v, iidx_v, urows_v, irows_v, out_v, usem, isem):
    wid = lax.axis_index("s") * NC + lax.axis_index("c")
    base = wid * BPW

    utab32 = utab_hbm.bitcast(jnp.int32)
    itab32 = itab_hbm.bitcast(jnp.int32)

    pltpu.sync_copy(uidx_hbm.at[pl.ds(base, BPW)], uidx_v)
    pltpu.sync_copy(iidx_hbm.at[pl.ds(base, BPW)], iidx_v)

    def chunk_body(ck, _):
        def fire(blk, _):
            uvec = (uidx_v[pl.ds(ck * CHUNK + blk * L, L)] & QMASK) >> 1
            ivec = (iidx_v[pl.ds(ck * CHUNK + blk * L, L)] & QMASK) >> 1
            for j in range(L):
                pltpu.async_copy(
                    utab32.at[pl.ds(uvec[j], 1)],
                    urows_v.at[pl.ds(blk * L + j, 1)], usem)
                pltpu.async_copy(
                    itab32.at[pl.ds(ivec[j], 1)],
                    irows_v.at[pl.ds(blk * L + j, 1)], isem)
            return 0

        lax.fori_loop(0, CHUNK // L, fire, 0)
        pltpu.make_async_copy(
            utab32.at[pl.ds(0, CHUNK)], urows_v, usem).wait()
        pltpu.make_async_copy(
            itab32.at[pl.ds(0, CHUNK)], irows_v, isem).wait()

        def block_body(blk, _):
            row_ids = blk * L + lax.iota(jnp.int32, L)
            sl = pl.ds(ck * CHUNK + blk * L, L)
            uq = uidx_v[sl]
            iq = iidx_v[sl]
            ucol0 = (uq >> 19) * D
            icol0 = (iq >> 19) * D
            upsh = (uq & 1) * 16
            ipsh = (iq & 1) * 16
            sh16 = jnp.full((L,), 16, jnp.int32)

            def d_body(jj, carry):
                dot, uu, ii = carry
                uw = plsc.load_gather(urows_v, [row_ids, ucol0 + jj])
                iw = plsc.load_gather(irows_v, [row_ids, icol0 + jj])
                u = plsc.bitcast(
                    lax.shift_left(lax.shift_right_logical(uw, upsh), sh16),
                    jnp.float32)
                v = plsc.bitcast(
                    lax.shift_left(lax.shift_right_logical(iw, ipsh), sh16),
                    jnp.float32)
                return (dot + u * v, uu + u * u, ii + v * v)

            z = jnp.zeros((L,), jnp.float32)
            dot, uu, ii = lax.fori_loop(0, D, d_body, (z, z, z))

            p = jnp.maximum(uu * ii, 1e-30)
            bits = plsc.bitcast(p, jnp.int32)
            seed = jnp.full((L,), 0x5F3759DF, jnp.int32) - lax.shift_right_logical(
                bits, jnp.full((L,), 1, jnp.int32))
            y = plsc.bitcast(seed, jnp.float32)
            for _ in range(3):
                y = y * (1.5 - 0.5 * p * y * y)
            s = p * y
            denom = jnp.maximum(s, 1e-8)
            out_v[pl.ds(ck * CHUNK + blk * L, L)] = dot / denom
            return 0

        lax.fori_loop(0, CHUNK // L, block_body, 0)
        return 0

    lax.fori_loop(0, BPW // CHUNK, chunk_body, 0)
    pltpu.sync_copy(out_v, out_hbm.at[pl.ds(base, BPW)])


@jax.jit
def _run(query_users, query_items, user_table, item_table):
    tc_transpose = _make_tc_transpose()
    u_rows_tab = tc_transpose(user_table.T, user_table.T)
    i_rows_tab = tc_transpose(item_table.T, item_table.T)

    mesh = plsc.VectorSubcoreMesh(core_axis_name="c", subcore_axis_name="s")
    k = functools.partial(
        pl.kernel,
        mesh=mesh,
        compiler_params=pltpu.CompilerParams(needs_layout_passes=False),
        out_type=jax.ShapeDtypeStruct((B,), jnp.float32),
        scratch_types=[
            pltpu.VMEM((BPW,), jnp.int32),
            pltpu.VMEM((BPW,), jnp.int32),
            pltpu.VMEM((CHUNK, 128), jnp.int32),
            pltpu.VMEM((CHUNK, 128), jnp.int32),
            pltpu.VMEM((BPW,), jnp.float32),
            pltpu.SemaphoreType.DMA,
            pltpu.SemaphoreType.DMA,
        ],
    )(_cosine_body)
    return k(query_users, query_items, u_rows_tab, i_rows_tab)


def kernel(query_users, query_items, user_table, item_table):
    qu = query_users.astype(jnp.int32)
    qi = query_items.astype(jnp.int32)
    return _run(qu, qi, user_table, item_table)


# speedup vs baseline: 1.7035x; 1.3398x over previous
import functools

import jax
import jax.numpy as jnp
from jax import lax
from jax.experimental import pallas as pl
from jax.experimental.pallas import tpu as pltpu
from jax.experimental.pallas import tpu_sc as plsc

B = 16384
D = 64
NROWS = 1000000
L = 16
NC = 2
NS = 16
NW = NC * NS
BPW = B // NW
CHUNK = 256

QROWS = 524288
QMASK = QROWS - 1
QBLK = 4096
TGRID = QROWS // QBLK


def _transpose_body(x0_ref, x1_ref, y_ref):
    eye = jnp.eye(2 * D, dtype=jnp.float32)
    x = jnp.concatenate([x0_ref[...], x1_ref[...]], axis=0)
    y = lax.dot_general(
        x, eye, (((0,), (0,)), ((), ())),
        precision=lax.Precision.DEFAULT)
    y_ref[...] = y.astype(jnp.bfloat16)


def _make_tc_transpose():
    last_blk = NROWS // QBLK
    specs = [
        pl.BlockSpec(
            (D, QBLK),
            functools.partial(
                lambda k, i: (0, jnp.minimum(TGRID * k + i, last_blk)), k))
        for k in range(2)
    ]
    return pl.pallas_call(
        _transpose_body,
        grid=(TGRID,),
        in_specs=specs,
        out_specs=pl.BlockSpec((QBLK, 2 * D), lambda i: (i, 0)),
        out_shape=jax.ShapeDtypeStruct((QROWS, 2 * D), jnp.bfloat16),
    )


def _cosine_body(uidx_hbm, iidx_hbm, utab_hbm, itab_hbm, out_hbm,
                 uidx_v, iidx_v, urows_v, irows_v, out_v, usem, isem):
    wid = lax.axis_index("s") * NC + lax.axis_index("c")
    base = wid * BPW

    utab32 = utab_hbm.bitcast(jnp.int32)
    itab32 = itab_hbm.bitcast(jnp.int32)

    pltpu.sync_copy(uidx_hbm.at[pl.ds(base, BPW)], uidx_v)
    pltpu.sync_copy(iidx_hbm.at[pl.ds(base, BPW)], iidx_v)

    def chunk_body(ck, _):
        def fire(blk, _):
            uvec = (uidx_v[pl.ds(ck * CHUNK + blk * L, L)] & QMASK) >> 1
            ivec = (iidx_v[pl.ds(ck * CHUNK + blk * L, L)] & QMASK) >> 1
            for j in range(L):
                pltpu.async_copy(
                    utab32.at[pl.ds(uvec[j], 1)],
                    urows_v.at[pl.ds(blk * L + j, 1)], usem)
                pltpu.async_copy(
                    itab32.at[pl.ds(ivec[j], 1)],
                    irows_v.at[pl.ds(blk * L + j, 1)], isem)
            return 0

        lax.fori_loop(0, CHUNK // L, fire, 0)
        pltpu.make_async_copy(
            utab32.at[pl.ds(0, CHUNK)], urows_v, usem).wait()
        pltpu.make_async_copy(
            itab32.at[pl.ds(0, CHUNK)], irows_v, isem).wait()

        def block_body(blk, _):
            row_ids = blk * L + lax.iota(jnp.int32, L)
            sl = pl.ds(ck * CHUNK + blk * L, L)
            uq = uidx_v[sl]
            iq = iidx_v[sl]
            ucol0 = (uq >> 19) * D
            icol0 = (iq >> 19) * D
            upsh = (uq & 1) * 16
            ipsh = (iq & 1) * 16
            sh16 = jnp.full((L,), 16, jnp.int32)

            def d_body(jj, carry):
                dot, uu, ii = carry
                uw = plsc.load_gather(urows_v, [row_ids, ucol0 + jj])
                iw = plsc.load_gather(irows_v, [row_ids, icol0 + jj])
                u = plsc.bitcast(
                    lax.shift_left(lax.shift_right_logical(uw, upsh), sh16),
                    jnp.float32)
                v = plsc.bitcast(
                    lax.shift_left(lax.shift_right_logical(iw, ipsh), sh16),
                    jnp.float32)
                return (dot + u * v, uu + u * u, ii + v * v)

            z = jnp.zeros((L,), jnp.float32)
            dot, uu, ii = lax.fori_loop(0, D, d_body, (z, z, z))

            p = jnp.maximum(uu * ii, 1e-30)
            bits = plsc.bitcast(p, jnp.int32)
            seed = jnp.full((L,), 0x5F3759DF, jnp.int32) - lax.shift_right_logical(
                bits, jnp.full((L,), 1, jnp.int32))
            y = plsc.bitcast(seed, jnp.float32)
            for _ in range(3):
                y = y * (1.5 - 0.5 * p * y * y)
            s = p * y
            denom = jnp.maximum(s, 1e-8)
            out_v[pl.ds(ck * CHUNK + blk * L, L)] = dot / denom
            return 0

        lax.fori_loop(0, CHUNK // L, block_body, 0)
        return 0

    lax.fori_loop(0, BPW // CHUNK, chunk_body, 0)
    pltpu.sync_copy(out_v, out_hbm.at[pl.ds(base, BPW)])


@jax.jit
def _run(query_users, query_items, user_table, item_table):
    tc_transpose = _make_tc_transpose()
    u_rows_tab = tc_transpose(user_table.T, user_table.T)
    i_rows_tab = tc_transpose(item_table.T, item_table.T)

    mesh = plsc.VectorSubcoreMesh(core_axis_name="c", subcore_axis_name="s")
    k = functools.partial(
        pl.kernel,
        mesh=mesh,
        compiler_params=pltpu.CompilerParams(needs_layout_passes=False),
        out_type=jax.ShapeDtypeStruct((B,), jnp.float32),
        scratch_types=[
            pltpu.VMEM((BPW,), jnp.int32),
            pltpu.VMEM((BPW,), jnp.int32),
            pltpu.VMEM((CHUNK, 128), jnp.int32),
            pltpu.VMEM((CHUNK, 128), jnp.int32),
            pltpu.VMEM((BPW,), jnp.float32),
            pltpu.SemaphoreType.DMA,
            pltpu.SemaphoreType.DMA,
        ],
    )(_cosine_body)
    return k(query_users, query_items, u_rows_tab, i_rows_tab)


def kernel(query_users, query_items, user_table, item_table):
    qu = query_users.astype(jnp.int32)
    qi = query_items.astype(jnp.int32)
    return _run(qu, qi, user_table, item_table)


# speedup vs baseline: 2.1209x; 1.2450x over previous
import functools

import jax
import jax.numpy as jnp
from jax import lax
from jax.experimental import pallas as pl
from jax.experimental.pallas import tpu as pltpu
from jax.experimental.pallas import tpu_sc as plsc

B = 16384
D = 64
NROWS = 1000000
L = 16
NC = 2
NS = 16
NW = NC * NS
BPW = B // NW
CHUNK = 256

QROWS = 524288
QMASK = QROWS - 1
QBLK = 8192
TGRID = QROWS // QBLK


def _transpose_body(x0_ref, x1_ref, y_ref):
    eye = jnp.eye(2 * D, dtype=jnp.float32)
    x = jnp.concatenate([x0_ref[...], x1_ref[...]], axis=0)
    y = lax.dot_general(
        x, eye, (((0,), (0,)), ((), ())),
        precision=lax.Precision.DEFAULT)
    y_ref[...] = y.astype(jnp.bfloat16)


def _make_tc_transpose():
    last_blk = NROWS // QBLK
    specs = [
        pl.BlockSpec(
            (D, QBLK),
            functools.partial(
                lambda k, i: (0, jnp.minimum(TGRID * k + i, last_blk)), k))
        for k in range(2)
    ]
    return pl.pallas_call(
        _transpose_body,
        grid=(TGRID,),
        in_specs=specs,
        out_specs=pl.BlockSpec((QBLK, 2 * D), lambda i: (i, 0)),
        out_shape=jax.ShapeDtypeStruct((QROWS, 2 * D), jnp.bfloat16),
    )


def _cosine_body(uidx_hbm, iidx_hbm, utab_hbm, itab_hbm, out_hbm,
                 uidx_v, iidx_v, urows_v, irows_v, out_v, usem, isem):
    wid = lax.axis_index("s") * NC + lax.axis_index("c")
    base = wid * BPW

    utab32 = utab_hbm.bitcast(jnp.int32)
    itab32 = itab_hbm.bitcast(jnp.int32)

    pltpu.sync_copy(uidx_hbm.at[pl.ds(base, BPW)], uidx_v)
    pltpu.sync_copy(iidx_hbm.at[pl.ds(base, BPW)], iidx_v)

    def chunk_body(ck, _):
        def fire(blk, _):
            uvec = (uidx_v[pl.ds(ck * CHUNK + blk * L, L)] & QMASK) >> 1
            ivec = (iidx_v[pl.ds(ck * CHUNK + blk * L, L)] & QMASK) >> 1
            for j in range(L):
                pltpu.async_copy(
                    utab32.at[pl.ds(uvec[j], 1)],
                    urows_v.at[pl.ds(blk * L + j, 1)], usem)
                pltpu.async_copy(
                    itab32.at[pl.ds(ivec[j], 1)],
                    irows_v.at[pl.ds(blk * L + j, 1)], isem)
            return 0

        lax.fori_loop(0, CHUNK // L, fire, 0)
        pltpu.make_async_copy(
            utab32.at[pl.ds(0, CHUNK)], urows_v, usem).wait()
        pltpu.make_async_copy(
            itab32.at[pl.ds(0, CHUNK)], irows_v, isem).wait()

        def block_body(blk, _):
            row_ids = blk * L + lax.iota(jnp.int32, L)
            sl = pl.ds(ck * CHUNK + blk * L, L)
            uq = uidx_v[sl]
            iq = iidx_v[sl]
            ucol0 = (uq >> 19) * D
            icol0 = (iq >> 19) * D
            upsh = (uq & 1) * 16
            ipsh = (iq & 1) * 16
            sh16 = jnp.full((L,), 16, jnp.int32)

            def d_body(jj, carry):
                dot, uu, ii = carry
                uw = plsc.load_gather(urows_v, [row_ids, ucol0 + jj])
                iw = plsc.load_gather(irows_v, [row_ids, icol0 + jj])
                u = plsc.bitcast(
                    lax.shift_left(lax.shift_right_logical(uw, upsh), sh16),
                    jnp.float32)
                v = plsc.bitcast(
                    lax.shift_left(lax.shift_right_logical(iw, ipsh), sh16),
                    jnp.float32)
                return (dot + u * v, uu + u * u, ii + v * v)

            z = jnp.zeros((L,), jnp.float32)
            dot, uu, ii = lax.fori_loop(0, D, d_body, (z, z, z))

            p = jnp.maximum(uu * ii, 1e-30)
            bits = plsc.bitcast(p, jnp.int32)
            seed = jnp.full((L,), 0x5F3759DF, jnp.int32) - lax.shift_right_logical(
                bits, jnp.full((L,), 1, jnp.int32))
            y = plsc.bitcast(seed, jnp.float32)
            for _ in range(3):
                y = y * (1.5 - 0.5 * p * y * y)
            s = p * y
            denom = jnp.maximum(s, 1e-8)
            out_v[pl.ds(ck * CHUNK + blk * L, L)] = dot / denom
            return 0

        lax.fori_loop(0, CHUNK // L, block_body, 0)
        return 0

    lax.fori_loop(0, BPW // CHUNK, chunk_body, 0)
    pltpu.sync_copy(out_v, out_hbm.at[pl.ds(base, BPW)])


@jax.jit
def _run(query_users, query_items, user_table, item_table):
    tc_transpose = _make_tc_transpose()
    u_rows_tab = tc_transpose(user_table.T, user_table.T)
    i_rows_tab = tc_transpose(item_table.T, item_table.T)

    mesh = plsc.VectorSubcoreMesh(core_axis_name="c", subcore_axis_name="s")
    k = functools.partial(
        pl.kernel,
        mesh=mesh,
        compiler_params=pltpu.CompilerParams(needs_layout_passes=False),
        out_type=jax.ShapeDtypeStruct((B,), jnp.float32),
        scratch_types=[
            pltpu.VMEM((BPW,), jnp.int32),
            pltpu.VMEM((BPW,), jnp.int32),
            pltpu.VMEM((CHUNK, 128), jnp.int32),
            pltpu.VMEM((CHUNK, 128), jnp.int32),
            pltpu.VMEM((BPW,), jnp.float32),
            pltpu.SemaphoreType.DMA,
            pltpu.SemaphoreType.DMA,
        ],
    )(_cosine_body)
    return k(query_users, query_items, u_rows_tab, i_rows_tab)


def kernel(query_users, query_items, user_table, item_table):
    qu = query_users.astype(jnp.int32)
    qi = query_items.astype(jnp.int32)
    return _run(qu, qi, user_table, item_table)


# speedup vs baseline: 2.2124x; 1.0431x over previous
import functools

import jax
import jax.numpy as jnp
from jax import lax
from jax.experimental import pallas as pl
from jax.experimental.pallas import tpu as pltpu
from jax.experimental.pallas import tpu_sc as plsc

B = 16384
D = 64
NROWS = 1000000
L = 16
NC = 2
NS = 16
NW = NC * NS
BPW = B // NW
CHUNK = 256

QROWS = 524288
QMASK = QROWS - 1
QBLK = 16384
TGRID = QROWS // QBLK


def _transpose_body(x0_ref, x1_ref, y_ref):
    eye = jnp.eye(2 * D, dtype=jnp.float32)
    x = jnp.concatenate([x0_ref[...], x1_ref[...]], axis=0)
    y = lax.dot_general(
        x, eye, (((0,), (0,)), ((), ())),
        precision=lax.Precision.DEFAULT)
    y_ref[...] = y.astype(jnp.bfloat16)


def _make_tc_transpose():
    last_blk = NROWS // QBLK
    specs = [
        pl.BlockSpec(
            (D, QBLK),
            functools.partial(
                lambda k, i: (0, jnp.minimum(TGRID * k + i, last_blk)), k))
        for k in range(2)
    ]
    return pl.pallas_call(
        _transpose_body,
        grid=(TGRID,),
        in_specs=specs,
        out_specs=pl.BlockSpec((QBLK, 2 * D), lambda i: (i, 0)),
        out_shape=jax.ShapeDtypeStruct((QROWS, 2 * D), jnp.bfloat16),
    )


def _cosine_body(uidx_hbm, iidx_hbm, utab_hbm, itab_hbm, out_hbm,
                 uidx_v, iidx_v, urows_v, irows_v, out_v, usem, isem):
    wid = lax.axis_index("s") * NC + lax.axis_index("c")
    base = wid * BPW

    utab32 = utab_hbm.bitcast(jnp.int32)
    itab32 = itab_hbm.bitcast(jnp.int32)

    pltpu.sync_copy(uidx_hbm.at[pl.ds(base, BPW)], uidx_v)
    pltpu.sync_copy(iidx_hbm.at[pl.ds(base, BPW)], iidx_v)

    def chunk_body(ck, _):
        def fire(blk, _):
            uvec = (uidx_v[pl.ds(ck * CHUNK + blk * L, L)] & QMASK) >> 1
            ivec = (iidx_v[pl.ds(ck * CHUNK + blk * L, L)] & QMASK) >> 1
            for j in range(L):
                pltpu.async_copy(
                    utab32.at[pl.ds(uvec[j], 1)],
                    urows_v.at[pl.ds(blk * L + j, 1)], usem)
                pltpu.async_copy(
                    itab32.at[pl.ds(ivec[j], 1)],
                    irows_v.at[pl.ds(blk * L + j, 1)], isem)
            return 0

        lax.fori_loop(0, CHUNK // L, fire, 0)
        pltpu.make_async_copy(
            utab32.at[pl.ds(0, CHUNK)], urows_v, usem).wait()
        pltpu.make_async_copy(
            itab32.at[pl.ds(0, CHUNK)], irows_v, isem).wait()

        def block_body(blk, _):
            row_ids = blk * L + lax.iota(jnp.int32, L)
            sl = pl.ds(ck * CHUNK + blk * L, L)
            uq = uidx_v[sl]
            iq = iidx_v[sl]
            ucol0 = (uq >> 19) * D
            icol0 = (iq >> 19) * D
            upsh = (uq & 1) * 16
            ipsh = (iq & 1) * 16
            sh16 = jnp.full((L,), 16, jnp.int32)

            def d_body(jj, carry):
                dot, uu, ii = carry
                uw = plsc.load_gather(urows_v, [row_ids, ucol0 + jj])
                iw = plsc.load_gather(irows_v, [row_ids, icol0 + jj])
                u = plsc.bitcast(
                    lax.shift_left(lax.shift_right_logical(uw, upsh), sh16),
                    jnp.float32)
                v = plsc.bitcast(
                    lax.shift_left(lax.shift_right_logical(iw, ipsh), sh16),
                    jnp.float32)
                return (dot + u * v, uu + u * u, ii + v * v)

            z = jnp.zeros((L,), jnp.float32)
            dot, uu, ii = lax.fori_loop(0, D, d_body, (z, z, z))

            p = jnp.maximum(uu * ii, 1e-30)
            bits = plsc.bitcast(p, jnp.int32)
            seed = jnp.full((L,), 0x5F3759DF, jnp.int32) - lax.shift_right_logical(
                bits, jnp.full((L,), 1, jnp.int32))
            y = plsc.bitcast(seed, jnp.float32)
            for _ in range(3):
                y = y * (1.5 - 0.5 * p * y * y)
            s = p * y
            denom = jnp.maximum(s, 1e-8)
            out_v[pl.ds(ck * CHUNK + blk * L, L)] = dot / denom
            return 0

        lax.fori_loop(0, CHUNK // L, block_body, 0)
        return 0

    lax.fori_loop(0, BPW // CHUNK, chunk_body, 0)
    pltpu.sync_copy(out_v, out_hbm.at[pl.ds(base, BPW)])


@jax.jit
def _run(query_users, query_items, user_table, item_table):
    tc_transpose = _make_tc_transpose()
    u_rows_tab = tc_transpose(user_table.T, user_table.T)
    i_rows_tab = tc_transpose(item_table.T, item_table.T)

    mesh = plsc.VectorSubcoreMesh(core_axis_name="c", subcore_axis_name="s")
    k = functools.partial(
        pl.kernel,
        mesh=mesh,
        compiler_params=pltpu.CompilerParams(needs_layout_passes=False),
        out_type=jax.ShapeDtypeStruct((B,), jnp.float32),
        scratch_types=[
            pltpu.VMEM((BPW,), jnp.int32),
            pltpu.VMEM((BPW,), jnp.int32),
            pltpu.VMEM((CHUNK, 128), jnp.int32),
            pltpu.VMEM((CHUNK, 128), jnp.int32),
            pltpu.VMEM((BPW,), jnp.float32),
            pltpu.SemaphoreType.DMA,
            pltpu.SemaphoreType.DMA,
        ],
    )(_cosine_body)
    return k(query_users, query_items, u_rows_tab, i_rows_tab)


def kernel(query_users, query_items, user_table, item_table):
    qu = query_users.astype(jnp.int32)
    qi = query_items.astype(jnp.int32)
    return _run(qu, qi, user_table, item_table)


# speedup vs baseline: 2.3153x; 1.0465x over previous
import functools

import jax
import jax.numpy as jnp
from jax import lax
from jax.experimental import pallas as pl
from jax.experimental.pallas import tpu as pltpu
from jax.experimental.pallas import tpu_sc as plsc

B = 16384
D = 64
NROWS = 1000000
L = 16
NC = 2
NS = 16
NW = NC * NS
BPW = B // NW
CHUNK = 256

QROWS = 524288
QMASK = QROWS - 1
QBLK = 32768
TGRID = QROWS // QBLK


def _transpose_body(x0_ref, x1_ref, y_ref):
    eye = jnp.eye(2 * D, dtype=jnp.float32)
    x = jnp.concatenate([x0_ref[...], x1_ref[...]], axis=0)
    y = lax.dot_general(
        x, eye, (((0,), (0,)), ((), ())),
        precision=lax.Precision.DEFAULT)
    y_ref[...] = y.astype(jnp.bfloat16)


def _make_tc_transpose():
    last_blk = NROWS // QBLK
    specs = [
        pl.BlockSpec(
            (D, QBLK),
            functools.partial(
                lambda k, i: (0, jnp.minimum(TGRID * k + i, last_blk)), k))
        for k in range(2)
    ]
    return pl.pallas_call(
        _transpose_body,
        grid=(TGRID,),
        in_specs=specs,
        out_specs=pl.BlockSpec((QBLK, 2 * D), lambda i: (i, 0)),
        out_shape=jax.ShapeDtypeStruct((QROWS, 2 * D), jnp.bfloat16),
    )


def _cosine_body(uidx_hbm, iidx_hbm, utab_hbm, itab_hbm, out_hbm,
                 uidx_v, iidx_v, urows_v, irows_v, out_v, usem, isem):
    wid = lax.axis_index("s") * NC + lax.axis_index("c")
    base = wid * BPW

    utab32 = utab_hbm.bitcast(jnp.int32)
    itab32 = itab_hbm.bitcast(jnp.int32)

    pltpu.sync_copy(uidx_hbm.at[pl.ds(base, BPW)], uidx_v)
    pltpu.sync_copy(iidx_hbm.at[pl.ds(base, BPW)], iidx_v)

    def chunk_body(ck, _):
        def fire(blk, _):
            uvec = (uidx_v[pl.ds(ck * CHUNK + blk * L, L)] & QMASK) >> 1
            ivec = (iidx_v[pl.ds(ck * CHUNK + blk * L, L)] & QMASK) >> 1
            for j in range(L):
                pltpu.async_copy(
                    utab32.at[pl.ds(uvec[j], 1)],
                    urows_v.at[pl.ds(blk * L + j, 1)], usem)
                pltpu.async_copy(
                    itab32.at[pl.ds(ivec[j], 1)],
                    irows_v.at[pl.ds(blk * L + j, 1)], isem)
            return 0

        lax.fori_loop(0, CHUNK // L, fire, 0)
        pltpu.make_async_copy(
            utab32.at[pl.ds(0, CHUNK)], urows_v, usem).wait()
        pltpu.make_async_copy(
            itab32.at[pl.ds(0, CHUNK)], irows_v, isem).wait()

        def block_body(blk, _):
            row_ids = blk * L + lax.iota(jnp.int32, L)
            sl = pl.ds(ck * CHUNK + blk * L, L)
            uq = uidx_v[sl]
            iq = iidx_v[sl]
            ucol0 = (uq >> 19) * D
            icol0 = (iq >> 19) * D
            upsh = (uq & 1) * 16
            ipsh = (iq & 1) * 16
            sh16 = jnp.full((L,), 16, jnp.int32)

            def d_body(jj, carry):
                dot, uu, ii = carry
                uw = plsc.load_gather(urows_v, [row_ids, ucol0 + jj])
                iw = plsc.load_gather(irows_v, [row_ids, icol0 + jj])
                u = plsc.bitcast(
                    lax.shift_left(lax.shift_right_logical(uw, upsh), sh16),
                    jnp.float32)
                v = plsc.bitcast(
                    lax.shift_left(lax.shift_right_logical(iw, ipsh), sh16),
                    jnp.float32)
                return (dot + u * v, uu + u * u, ii + v * v)

            z = jnp.zeros((L,), jnp.float32)
            dot, uu, ii = lax.fori_loop(0, D, d_body, (z, z, z))

            p = jnp.maximum(uu * ii, 1e-30)
            bits = plsc.bitcast(p, jnp.int32)
            seed = jnp.full((L,), 0x5F3759DF, jnp.int32) - lax.shift_right_logical(
                bits, jnp.full((L,), 1, jnp.int32))
            y = plsc.bitcast(seed, jnp.float32)
            for _ in range(3):
                y = y * (1.5 - 0.5 * p * y * y)
            s = p * y
            denom = jnp.maximum(s, 1e-8)
            out_v[pl.ds(ck * CHUNK + blk * L, L)] = dot / denom
            return 0

        lax.fori_loop(0, CHUNK // L, block_body, 0)
        return 0

    lax.fori_loop(0, BPW // CHUNK, chunk_body, 0)
    pltpu.sync_copy(out_v, out_hbm.at[pl.ds(base, BPW)])


@jax.jit
def _run(query_users, query_items, user_table, item_table):
    tc_transpose = _make_tc_transpose()
    u_rows_tab = tc_transpose(user_table.T, user_table.T)
    i_rows_tab = tc_transpose(item_table.T, item_table.T)

    mesh = plsc.VectorSubcoreMesh(core_axis_name="c", subcore_axis_name="s")
    k = functools.partial(
        pl.kernel,
        mesh=mesh,
        compiler_params=pltpu.CompilerParams(needs_layout_passes=False),
        out_type=jax.ShapeDtypeStruct((B,), jnp.float32),
        scratch_types=[
            pltpu.VMEM((BPW,), jnp.int32),
            pltpu.VMEM((BPW,), jnp.int32),
            pltpu.VMEM((CHUNK, 128), jnp.int32),
            pltpu.VMEM((CHUNK, 128), jnp.int32),
            pltpu.VMEM((BPW,), jnp.float32),
            pltpu.SemaphoreType.DMA,
            pltpu.SemaphoreType.DMA,
        ],
    )(_cosine_body)
    return k(query_users, query_items, u_rows_tab, i_rows_tab)


def kernel(query_users, query_items, user_table, item_table):
    qu = query_users.astype(jnp.int32)
    qi = query_items.astype(jnp.int32)
    return _run(qu, qi, user_table, item_table)
